# baseline (speedup 1.0000x reference)
import functools

import jax
import jax.numpy as jnp
from jax import lax
from jax.experimental import pallas as pl
from jax.experimental.pallas import tpu as pltpu
from jax.experimental.pallas import tpu_sc as plsc

N = 100000
D = 128
S = 512
NC = 2
NSUB = 16
NW = NC * NSUB
C = N // NW
WIN = 3136
XB = 112
NXB = WIN // XB
NGB = XB // 16
NV = D // 16
CHUNK = 3152
BPAD = 24
NEG_INF = float("-inf")

_mesh = plsc.VectorSubcoreMesh(
    core_axis_name="c", subcore_axis_name="s", num_cores=NC, num_subcores=NSUB
)


@functools.partial(
    pl.kernel,
    out_type=(
        jax.ShapeDtypeStruct((S * D,), jnp.float32),
        jax.ShapeDtypeStruct((S * D,), jnp.float32),
        jax.ShapeDtypeStruct((2 * NW * D,), jnp.float32),
        jax.ShapeDtypeStruct((2 * NW * D,), jnp.float32),
        jax.ShapeDtypeStruct((NW * 16,), jnp.int32),
    ),
    mesh=_mesh,
    scratch_types=(
        pltpu.VMEM((XB * D,), jnp.float32),
        pltpu.VMEM((XB * D,), jnp.float32),
        pltpu.VMEM((CHUNK,), jnp.int32),
        pltpu.VMEM((144,), jnp.float32),
        pltpu.VMEM((D,), jnp.float32),
        pltpu.VMEM((D,), jnp.float32),
        pltpu.VMEM((D,), jnp.float32),
        pltpu.VMEM((D,), jnp.float32),
        pltpu.VMEM((16,), jnp.int32),
        pltpu.VMEM((32,), jnp.float32),
        pltpu.VMEM((32, 17), jnp.float32),
        pltpu.SemaphoreType.DMA,
    ),
    compiler_params=pltpu.CompilerParams(needs_layout_passes=False),
)
def _phase1(x_hbm, batch_hbm, wb_hbm, out1_hbm, out2_hbm, side1_hbm,
            side2_hbm, sideseg_hbm, xblka, xblkb, bchunk, wbv, stg1, stg2,
            zrow, irow, segrow, sbuf, pmat, sem):
    w = lax.axis_index("s") * NC + lax.axis_index("c")
    row0 = w * C
    bbase = pl.multiple_of(row0 - lax.rem(row0, 8), 8)
    bskew = row0 - bbase
    astart = pl.multiple_of(
        jnp.minimum(row0 - lax.rem(row0, 8), N - WIN), 8)
    skew = row0 - astart
    boff = bskew - skew

    pltpu.sync_copy(wb_hbm, wbv)
    pltpu.sync_copy(batch_hbm.at[pl.ds(bbase, CHUNK)], bchunk)

    zero16 = jnp.zeros((16,), jnp.float32)
    ninf16 = jnp.full((16,), NEG_INF, jnp.float32)
    for k in range(NV):
        sl = pl.ds(k * 16, 16)
        zrow[sl] = zero16
        irow[sl] = ninf16

    def iget(idx):
        return bchunk[pl.ds(idx, 16)][0]

    wvecs = tuple(wbv[pl.ds(k * 16, 16)] for k in range(NV))
    bscal = wbv[pl.ds(128, 16)][0]
    first_seg = iget(bskew)
    next_first = iget(bskew + C)

    def stage_runs(run1, run2):
        for k in range(NV):
            sl = pl.ds(k * 16, 16)
            stg1[sl] = run1[k]
            stg2[sl] = run2[k]

    def rowslice(seg):
        return pl.ds(pl.multiple_of(seg * D, 8), D)

    def flush_out(seg):
        pltpu.sync_copy(stg1, out1_hbm.at[rowslice(seg)])
        pltpu.sync_copy(stg2, out2_hbm.at[rowslice(seg)])

    def flush_side(entry):
        pltpu.sync_copy(stg1, side1_hbm.at[rowslice(entry)])
        pltpu.sync_copy(stg2, side2_hbm.at[rowslice(entry)])

    def init_gaps(lo, hi):
        def body(g, _):
            pltpu.sync_copy(zrow, out1_hbm.at[rowslice(g)])
            pltpu.sync_copy(irow, out2_hbm.at[rowslice(g)])
            return 0
        lax.fori_loop(lo, hi, body, 0)

    zeros8 = (zero16,) * NV
    ninfs8 = (ninf16,) * NV
    lane = lax.iota(jnp.int32, 16)

    def rowpass(buf, base_l, par):
        gbase = base_l * D
        for r in range(16):
            rb = gbase + r * D
            xv = tuple(buf[pl.ds(rb + k * 16, 16)] for k in range(NV))
            m = [xv[k] * wvecs[k] for k in range(NV)]
            pr = ((m[0] + m[1]) + (m[2] + m[3])) + \
                 ((m[4] + m[5]) + (m[6] + m[7]))
            pmat[par * 16 + r, pl.ds(0, 16)] = pr

    def get_scores(par):
        rows16 = jnp.full((16,), par * 16, jnp.int32) + lane
        accs = [jnp.zeros((16,), jnp.float32) for _ in range(4)]
        for l in range(16):
            colv = plsc.load_gather(
                pmat, [rows16, jnp.full((16,), l, jnp.int32)])
            accs[l % 4] = accs[l % 4] + colv
        acc = (accs[0] + accs[1]) + (accs[2] + accs[3])
        z = acc + jnp.full((16,), bscal, jnp.float32)
        return 1.0 / (1.0 + jnp.exp(-z))

    def accum16(buf, base_l, score, run1, run2):
        gbase = base_l * D
        for r in range(16):
            rb = gbase + r * D
            xv = tuple(buf[pl.ds(rb + k * 16, 16)] for k in range(NV))
            sc = jnp.full((16,), score[r], jnp.float32)
            run1 = tuple(run1[k] + sc * xv[k] for k in range(NV))
            run2 = tuple(jnp.maximum(run2[k], xv[k]) for k in range(NV))
        return run1, run2

    def slow_group(buf, gwlo, blkstart, lo, hi, score, carry):
        sbuf[pl.ds(0, 16)] = score

        def srow(i, carry):
            cur_seg = carry[0]
            run1 = carry[1:1 + NV]
            run2 = carry[1 + NV:]
            seg = iget(boff + i)

            def on_change(_):
                stage_runs(run1, run2)
                lax.cond(
                    cur_seg == first_seg,
                    lambda _: (flush_side(2 * w), 0)[1],
                    lambda _: (flush_out(cur_seg), 0)[1],
                    0,
                )
                init_gaps(cur_seg + 1, seg)
                return (seg,) + zeros8 + ninfs8

            carry2 = lax.cond(seg != cur_seg, on_change, lambda _: carry, 0)
            cur2 = carry2[0]
            r1 = carry2[1:1 + NV]
            r2 = carry2[1 + NV:]
            lr = (i - blkstart) * D
            xv = tuple(buf[pl.ds(lr + k * 16, 16)] for k in range(NV))
            sc = jnp.full((16,), sbuf[pl.ds(i - gwlo, 16)][0], jnp.float32)
            r1 = tuple(r1[k] + sc * xv[k] for k in range(NV))
            r2 = tuple(jnp.maximum(r2[k], xv[k]) for k in range(NV))
            return (cur2,) + r1 + r2

        return lax.fori_loop(lo, hi, srow, carry)

    def process_group(buf, blkstart, gl, score, carry):
        base_l = gl * 16
        gwlo = blkstart + base_l
        bvo = jnp.maximum(boff + gwlo, 0)
        cur_seg = carry[0]
        s_lo = iget(bvo)
        s_hi = iget(bvo + 15)
        full = (gwlo >= skew) & (gwlo + 16 <= skew + C)
        takefast = full & (s_lo == s_hi) & (s_lo == cur_seg)

        def fast(carry):
            run1, run2 = accum16(buf, base_l, score,
                                 carry[1:1 + NV], carry[1 + NV:])
            return (carry[0],) + run1 + run2

        def slow(carry):
            lo = jnp.maximum(gwlo, skew)
            hi = jnp.minimum(gwlo + 16, skew + C)
            return slow_group(buf, gwlo, blkstart, lo, hi, score, carry)

        return lax.cond(takefast, fast, slow, carry)

    def blk_process(buf, bi, carry):
        blkstart = bi * XB
        rowpass(buf, 0, jnp.int32(0))

        def gbody(gl, carry):
            par = lax.rem(gl, 2)
            score = get_scores(par)
            rowpass(buf, (gl + 1) * 16, 1 - par)
            return process_group(buf, blkstart, gl, score, carry)

        carry = lax.fori_loop(0, NGB - 1, gbody, carry)
        score = get_scores(jnp.int32((NGB - 1) % 2))
        return process_group(buf, blkstart, NGB - 1, score, carry)

    def xoff(bi):
        return pl.multiple_of((astart + bi * XB) * D, 8)

    def start_copy(bi, buf):
        pltpu.async_copy(x_hbm.at[pl.ds(xoff(bi), XB * D)], buf, sem)

    def wait_copy(buf):
        pltpu.make_async_copy(x_hbm.at[pl.ds(0, XB * D)], buf, sem).wait()

    start_copy(0, xblka)

    def pair_body(p, carry):
        for b in range(2):
            buf = xblka if b == 0 else xblkb
            nbuf = xblkb if b == 0 else xblka
            bi = 2 * p + b
            wait_copy(buf)

            @pl.when(bi + 1 < NXB)
            def _():
                start_copy(bi + 1, nbuf)

            carry = blk_process(buf, bi, carry)
        return carry

    carry = (first_seg,) + zeros8 + ninfs8
    carry = lax.fori_loop(0, NXB // 2, pair_body, carry)

    cur_seg = carry[0]
    run1 = carry[1:1 + NV]
    run2 = carry[1 + NV:]

    stage_runs(run1, run2)

    def single_seg(_):
        flush_side(2 * w)
        pltpu.sync_copy(zrow, side1_hbm.at[rowslice(2 * w + 1)])
        pltpu.sync_copy(irow, side2_hbm.at[rowslice(2 * w + 1)])
        return 0

    def multi_seg(_):
        flush_side(2 * w + 1)
        return 0

    lax.cond(cur_seg == first_seg, single_seg, multi_seg, 0)

    init_gaps(cur_seg + 1, next_first)

    @pl.when(w == 0)
    def _():
        init_gaps(0, first_seg)

    segrow[...] = jnp.where(lane == 0, first_seg,
                            jnp.where(lane == 1, cur_seg, 0))
    pltpu.sync_copy(segrow,
                    sideseg_hbm.at[pl.ds(pl.multiple_of(w * 16, 8), 16)])


def _phase2_body(sideseg_ref, p1_ref, p2_ref, side1_ref, side2_ref,
                 o1_ref, o2_ref):
    iota = lax.broadcasted_iota(jnp.int32, (S, 1), 0)

    def mask_body(e, bm):
        sid = sideseg_ref[e // 2, e % 2]
        return jnp.maximum(bm, (iota == sid).astype(jnp.int32))

    bmi = lax.fori_loop(0, 2 * NW, mask_body, jnp.zeros((S, 1), jnp.int32))
    bm = bmi > 0
    a1 = jnp.where(bm, 0.0, p1_ref[...])
    a2 = jnp.where(bm, NEG_INF, p2_ref[...])

    def comb_body(e, carry):
        a1, a2 = carry
        sid = sideseg_ref[e // 2, e % 2]
        rm = iota == sid
        s1 = side1_ref[pl.ds(e, 1), :]
        s2 = side2_ref[pl.ds(e, 1), :]
        a1 = a1 + jnp.where(rm, s1, 0.0)
        a2 = jnp.where(rm, jnp.maximum(a2, s2), a2)
        return a1, a2

    a1, a2 = lax.fori_loop(0, 2 * NW, comb_body, (a1, a2))
    o1_ref[...] = a1
    o2_ref[...] = a2


def kernel(x, batch, W, b):
    x_flat = x.reshape(N * D)
    batch32 = batch.astype(jnp.int32)
    batch_p = jnp.concatenate(
        [batch32, jnp.full((BPAD,), S, jnp.int32)]
    )
    wb = jnp.concatenate(
        [W.reshape(D).astype(jnp.float32), b.astype(jnp.float32),
         jnp.zeros((15,), jnp.float32)]
    )
    p1, p2, side1, side2, sideseg = _phase1(x_flat, batch_p, wb)
    p1 = p1.reshape(S, D)
    p2 = p2.reshape(S, D)
    side1 = side1.reshape(2 * NW, D)
    side2 = side2.reshape(2 * NW, D)
    sideseg = sideseg.reshape(NW, 16)
    o1, o2 = pl.pallas_call(
        _phase2_body,
        in_specs=[
            pl.BlockSpec(memory_space=pltpu.SMEM),
            pl.BlockSpec(memory_space=pltpu.VMEM),
            pl.BlockSpec(memory_space=pltpu.VMEM),
            pl.BlockSpec(memory_space=pltpu.VMEM),
            pl.BlockSpec(memory_space=pltpu.VMEM),
        ],
        out_specs=[
            pl.BlockSpec(memory_space=pltpu.VMEM),
            pl.BlockSpec(memory_space=pltpu.VMEM),
        ],
        out_shape=[
            jax.ShapeDtypeStruct((S, D), jnp.float32),
            jax.ShapeDtypeStruct((S, D), jnp.float32),
        ],
    )(sideseg, p1, p2, side1, side2)
    return jnp.concatenate([o1, o2], axis=1)


# speedup vs baseline: 1.2673x; 1.2673x over previous
import functools

import jax
import jax.numpy as jnp
from jax import lax
from jax.experimental import pallas as pl
from jax.experimental.pallas import tpu as pltpu
from jax.experimental.pallas import tpu_sc as plsc

N = 100000
D = 128
S = 512
NC = 2
NSUB = 16
NW = NC * NSUB
C = N // NW
WIN = 3136
XB = 112
NXB = WIN // XB
NGB = XB // 16
NV = D // 16
CHUNK = 3152
BPAD = 24
NEG_INF = float("-inf")

_mesh = plsc.VectorSubcoreMesh(
    core_axis_name="c", subcore_axis_name="s", num_cores=NC, num_subcores=NSUB
)


@functools.partial(
    pl.kernel,
    out_type=(
        jax.ShapeDtypeStruct((S * D,), jnp.float32),
        jax.ShapeDtypeStruct((S * D,), jnp.float32),
        jax.ShapeDtypeStruct((2 * NW * D,), jnp.float32),
        jax.ShapeDtypeStruct((2 * NW * D,), jnp.float32),
        jax.ShapeDtypeStruct((NW * 16,), jnp.int32),
    ),
    mesh=_mesh,
    scratch_types=(
        pltpu.VMEM((XB * D,), jnp.float32),
        pltpu.VMEM((XB * D,), jnp.float32),
        pltpu.VMEM((CHUNK,), jnp.int32),
        pltpu.VMEM((144,), jnp.float32),
        pltpu.VMEM((D,), jnp.float32),
        pltpu.VMEM((D,), jnp.float32),
        pltpu.VMEM((D,), jnp.float32),
        pltpu.VMEM((D,), jnp.float32),
        pltpu.VMEM((16,), jnp.int32),
        pltpu.SemaphoreType.DMA,
    ),
    compiler_params=pltpu.CompilerParams(needs_layout_passes=False),
)
def _phase1(x_hbm, batch_hbm, wb_hbm, out1_hbm, out2_hbm, side1_hbm,
            side2_hbm, sideseg_hbm, xblka, xblkb, bchunk, wbv, stg1, stg2,
            zrow, irow, segrow, sem):
    w = lax.axis_index("s") * NC + lax.axis_index("c")
    row0 = w * C
    bbase = pl.multiple_of(row0 - lax.rem(row0, 8), 8)
    bskew = row0 - bbase
    astart = pl.multiple_of(
        jnp.minimum(row0 - lax.rem(row0, 8), N - WIN), 8)
    skew = row0 - astart
    boff = bskew - skew

    pltpu.sync_copy(wb_hbm, wbv)
    pltpu.sync_copy(batch_hbm.at[pl.ds(bbase, CHUNK)], bchunk)

    zero16 = jnp.zeros((16,), jnp.float32)
    ninf16 = jnp.full((16,), NEG_INF, jnp.float32)
    for k in range(NV):
        sl = pl.ds(k * 16, 16)
        zrow[sl] = zero16
        irow[sl] = ninf16

    def iget(idx):
        return bchunk[pl.ds(idx, 16)][0]

    wvecs = tuple(wbv[pl.ds(k * 16, 16)] for k in range(NV))
    bscal = wbv[pl.ds(128, 16)][0]
    first_seg = iget(bskew)
    next_first = iget(bskew + C)

    def stage_runs(run1, run2):
        for k in range(NV):
            sl = pl.ds(k * 16, 16)
            stg1[sl] = run1[k]
            stg2[sl] = run2[k]

    def rowslice(seg):
        return pl.ds(pl.multiple_of(seg * D, 8), D)

    def flush_out(seg):
        pltpu.sync_copy(stg1, out1_hbm.at[rowslice(seg)])
        pltpu.sync_copy(stg2, out2_hbm.at[rowslice(seg)])

    def flush_side(entry):
        pltpu.sync_copy(stg1, side1_hbm.at[rowslice(entry)])
        pltpu.sync_copy(stg2, side2_hbm.at[rowslice(entry)])

    def init_gaps(lo, hi):
        def body(g, _):
            pltpu.sync_copy(zrow, out1_hbm.at[rowslice(g)])
            pltpu.sync_copy(irow, out2_hbm.at[rowslice(g)])
            return 0
        lax.fori_loop(lo, hi, body, 0)

    zeros8 = (zero16,) * NV
    ninfs8 = (ninf16,) * NV
    lane = lax.iota(jnp.int32, 16)

    bsp = jnp.full((16,), bscal, jnp.float32)
    perms = tuple(jnp.bitwise_xor(lane, sh) for sh in (8, 4, 2, 1))

    def hsum_splat(v):
        for p in perms:
            v = v + v.at[p].get(mode="promise_in_bounds",
                                unique_indices=True)
        return v

    def fused_row(buf, rb, run1, run2):
        xv = tuple(buf[pl.ds(rb + k * 16, 16)] for k in range(NV))
        m = [xv[k] * wvecs[k] for k in range(NV)]
        pr = ((m[0] + m[1]) + (m[2] + m[3])) + \
             ((m[4] + m[5]) + (m[6] + m[7]))
        z = hsum_splat(pr) + bsp
        sc = 1.0 / (1.0 + jnp.exp(-z))
        run1 = tuple(run1[k] + sc * xv[k] for k in range(NV))
        run2 = tuple(jnp.maximum(run2[k], xv[k]) for k in range(NV))
        return run1, run2

    def accum16(buf, base_l, run1, run2):
        gbase = base_l * D
        for r in range(16):
            run1, run2 = fused_row(buf, gbase + r * D, run1, run2)
        return run1, run2

    def slow_group(buf, gwlo, blkstart, lo, hi, carry):
        def srow(i, carry):
            cur_seg = carry[0]
            run1 = carry[1:1 + NV]
            run2 = carry[1 + NV:]
            seg = iget(boff + i)

            def on_change(_):
                stage_runs(run1, run2)
                lax.cond(
                    cur_seg == first_seg,
                    lambda _: (flush_side(2 * w), 0)[1],
                    lambda _: (flush_out(cur_seg), 0)[1],
                    0,
                )
                init_gaps(cur_seg + 1, seg)
                return (seg,) + zeros8 + ninfs8

            carry2 = lax.cond(seg != cur_seg, on_change, lambda _: carry, 0)
            cur2 = carry2[0]
            r1 = carry2[1:1 + NV]
            r2 = carry2[1 + NV:]
            r1, r2 = fused_row(buf, (i - blkstart) * D, r1, r2)
            return (cur2,) + r1 + r2

        return lax.fori_loop(lo, hi, srow, carry)

    def make_group_body(buf, blkstart):
        def group_body(gl, carry):
            base_l = gl * 16
            gwlo = blkstart + base_l
            bvo = jnp.maximum(boff + gwlo, 0)
            cur_seg = carry[0]
            s_lo = iget(bvo)
            s_hi = iget(bvo + 15)
            full = (gwlo >= skew) & (gwlo + 16 <= skew + C)
            takefast = full & (s_lo == s_hi) & (s_lo == cur_seg)

            def fast(carry):
                run1, run2 = accum16(buf, base_l,
                                     carry[1:1 + NV], carry[1 + NV:])
                return (carry[0],) + run1 + run2

            def slow(carry):
                lo = jnp.maximum(gwlo, skew)
                hi = jnp.minimum(gwlo + 16, skew + C)
                return slow_group(buf, gwlo, blkstart, lo, hi, carry)

            return lax.cond(takefast, fast, slow, carry)

        return group_body

    def xoff(bi):
        return pl.multiple_of((astart + bi * XB) * D, 8)

    def start_copy(bi, buf):
        pltpu.async_copy(x_hbm.at[pl.ds(xoff(bi), XB * D)], buf, sem)

    def wait_copy(buf):
        pltpu.make_async_copy(x_hbm.at[pl.ds(0, XB * D)], buf, sem).wait()

    start_copy(0, xblka)

    def pair_body(p, carry):
        for b in range(2):
            buf = xblka if b == 0 else xblkb
            nbuf = xblkb if b == 0 else xblka
            bi = 2 * p + b
            wait_copy(buf)

            @pl.when(bi + 1 < NXB)
            def _():
                start_copy(bi + 1, nbuf)

            carry = lax.fori_loop(0, NGB, make_group_body(buf, bi * XB),
                                  carry)
        return carry

    carry = (first_seg,) + zeros8 + ninfs8
    carry = lax.fori_loop(0, NXB // 2, pair_body, carry)

    cur_seg = carry[0]
    run1 = carry[1:1 + NV]
    run2 = carry[1 + NV:]

    stage_runs(run1, run2)

    def single_seg(_):
        flush_side(2 * w)
        pltpu.sync_copy(zrow, side1_hbm.at[rowslice(2 * w + 1)])
        pltpu.sync_copy(irow, side2_hbm.at[rowslice(2 * w + 1)])
        return 0

    def multi_seg(_):
        flush_side(2 * w + 1)
        return 0

    lax.cond(cur_seg == first_seg, single_seg, multi_seg, 0)

    init_gaps(cur_seg + 1, next_first)

    @pl.when(w == 0)
    def _():
        init_gaps(0, first_seg)

    segrow[...] = jnp.where(lane == 0, first_seg,
                            jnp.where(lane == 1, cur_seg, 0))
    pltpu.sync_copy(segrow,
                    sideseg_hbm.at[pl.ds(pl.multiple_of(w * 16, 8), 16)])


def _phase2_body(sideseg_ref, p1_ref, p2_ref, side1_ref, side2_ref,
                 o1_ref, o2_ref):
    iota = lax.broadcasted_iota(jnp.int32, (S, 1), 0)

    def mask_body(e, bm):
        sid = sideseg_ref[e // 2, e % 2]
        return jnp.maximum(bm, (iota == sid).astype(jnp.int32))

    bmi = lax.fori_loop(0, 2 * NW, mask_body, jnp.zeros((S, 1), jnp.int32))
    bm = bmi > 0
    a1 = jnp.where(bm, 0.0, p1_ref[...])
    a2 = jnp.where(bm, NEG_INF, p2_ref[...])

    def comb_body(e, carry):
        a1, a2 = carry
        sid = sideseg_ref[e // 2, e % 2]
        rm = iota == sid
        s1 = side1_ref[pl.ds(e, 1), :]
        s2 = side2_ref[pl.ds(e, 1), :]
        a1 = a1 + jnp.where(rm, s1, 0.0)
        a2 = jnp.where(rm, jnp.maximum(a2, s2), a2)
        return a1, a2

    a1, a2 = lax.fori_loop(0, 2 * NW, comb_body, (a1, a2))
    o1_ref[...] = a1
    o2_ref[...] = a2


def kernel(x, batch, W, b):
    x_flat = x.reshape(N * D)
    batch32 = batch.astype(jnp.int32)
    batch_p = jnp.concatenate(
        [batch32, jnp.full((BPAD,), S, jnp.int32)]
    )
    wb = jnp.concatenate(
        [W.reshape(D).astype(jnp.float32), b.astype(jnp.float32),
         jnp.zeros((15,), jnp.float32)]
    )
    p1, p2, side1, side2, sideseg = _phase1(x_flat, batch_p, wb)
    p1 = p1.reshape(S, D)
    p2 = p2.reshape(S, D)
    side1 = side1.reshape(2 * NW, D)
    side2 = side2.reshape(2 * NW, D)
    sideseg = sideseg.reshape(NW, 16)
    o1, o2 = pl.pallas_call(
        _phase2_body,
        in_specs=[
            pl.BlockSpec(memory_space=pltpu.SMEM),
            pl.BlockSpec(memory_space=pltpu.VMEM),
            pl.BlockSpec(memory_space=pltpu.VMEM),
            pl.BlockSpec(memory_space=pltpu.VMEM),
            pl.BlockSpec(memory_space=pltpu.VMEM),
        ],
        out_specs=[
            pl.BlockSpec(memory_space=pltpu.VMEM),
            pl.BlockSpec(memory_space=pltpu.VMEM),
        ],
        out_shape=[
            jax.ShapeDtypeStruct((S, D), jnp.float32),
            jax.ShapeDtypeStruct((S, D), jnp.float32),
        ],
    )(sideseg, p1, p2, side1, side2)
    return jnp.concatenate([o1, o2], axis=1)
